# Initial kernel scaffold; baseline (speedup 1.0000x reference)
import jax
import jax.numpy as jnp
from jax.experimental import pallas as pl


def kernel(edge_index, W1_0, b1_0, W2_0, b2_0, W1_1, b1_1, W2_1, b2_1, Wp1, bp1, Wp2, bp2, Wv1, bv1, Wv2, bv2):
    raise NotImplementedError("write your pallas kernel here")


# speedup vs baseline: 56.8830x; 56.8830x over previous
import functools

import jax
import jax.numpy as jnp
from jax import lax
from jax.experimental import pallas as pl
from jax.experimental.pallas import tpu as pltpu
from jax.experimental.pallas import tpu_sc as plsc

N = 100000
E = 3200000
F = 16
NC = 2
NS = 16
NW = NC * NS
EPW = E // NW
K = 4000
NCHUNK = EPW // K
KA = 1000
NCHUNKA = EPW // KA
RPT = N // NS
DEG_T = 10
DEG_RPT = N // DEG_T
NP = N // 8

_mesh = plsc.VectorSubcoreMesh(core_axis_name="c", subcore_axis_name="s")
_sc_params = pltpu.CompilerParams(use_tc_tiling_on_sc=False)


@functools.partial(
    pl.kernel,
    out_type=jax.ShapeDtypeStruct((NC, N, 1), jnp.float32),
    mesh=_mesh,
    compiler_params=_sc_params,
    scratch_types=[
        pltpu.VMEM((K,), jnp.int32),
        pltpu.VMEM((K, 1), jnp.float32),
        pltpu.VMEM_SHARED((N, 1), jnp.float32),
        pltpu.SemaphoreType.DMA,
    ],
)
def _deg_kernel(half_hbm, ones_hbm, row_hbm, out_hbm, row_v, ones_v, acc, sem):
    c = lax.axis_index("c")
    s = lax.axis_index("s")
    wid = s * NC + c
    @pl.when(s < DEG_T)
    def _():
        pltpu.sync_copy(half_hbm.at[pl.ds(s * DEG_RPT, DEG_RPT)],
                        acc.at[pl.ds(s * DEG_RPT, DEG_RPT)])
    pltpu.sync_copy(ones_hbm, ones_v)
    plsc.subcore_barrier()

    def chunk(i, carry):
        off = wid * EPW + i * K
        pltpu.sync_copy(row_hbm.at[pl.ds(off, K)], row_v)
        pltpu.sync_copy(ones_v, acc.at[row_v], add=True)
        return carry

    lax.fori_loop(0, NCHUNK, chunk, 0)
    plsc.subcore_barrier()
    @pl.when(s < DEG_T)
    def _():
        pltpu.sync_copy(acc.at[pl.ds(s * DEG_RPT, DEG_RPT)],
                        out_hbm.at[c].at[pl.ds(s * DEG_RPT, DEG_RPT)])


@functools.partial(
    pl.kernel,
    out_type=jax.ShapeDtypeStruct((NC, N, F), jnp.float32),
    mesh=_mesh,
    compiler_params=_sc_params,
    scratch_types=[
        pltpu.VMEM((KA,), jnp.int32),
        pltpu.VMEM((KA,), jnp.int32),
        pltpu.VMEM((KA, F), jnp.float32),
        pltpu.VMEM_SHARED((N, F), jnp.float32),
        pltpu.SemaphoreType.DMA,
    ],
)
def _agg_kernel(x_hbm, xh_hbm, row_hbm, col_hbm, out_hbm,
                col_v, row_v, rows_v, acc, sem):
    c = lax.axis_index("c")
    s = lax.axis_index("s")
    wid = s * NC + c
    pltpu.sync_copy(xh_hbm.at[pl.ds(s * RPT, RPT)],
                    acc.at[pl.ds(s * RPT, RPT)])
    plsc.subcore_barrier()

    def chunk(i, carry):
        off = wid * EPW + i * KA
        pltpu.sync_copy(col_hbm.at[pl.ds(off, KA)], col_v)
        pltpu.async_copy(x_hbm.at[col_v], rows_v, sem).wait()
        pltpu.sync_copy(row_hbm.at[pl.ds(off, KA)], row_v)
        pltpu.sync_copy(rows_v, acc.at[row_v], add=True)
        return carry

    lax.fori_loop(0, NCHUNKA, chunk, 0)
    plsc.subcore_barrier()
    pltpu.sync_copy(acc.at[pl.ds(s * RPT, RPT)],
                    out_hbm.at[c].at[pl.ds(s * RPT, RPT)])


def _mlp0_body(dp_ref, s_ref, w1_ref, b1_ref, w2_ref, b2_ref, x1_ref):
    d8 = dp_ref[0] + dp_ref[1]
    db = jnp.dot(d8, s_ref[...], preferred_element_type=jnp.float32)
    h = jax.nn.relu(db * w1_ref[...] + b1_ref[...])
    x1_ref[...] = jax.nn.relu(
        jnp.dot(h, w2_ref[...], preferred_element_type=jnp.float32)
        + b2_ref[...])


def _mlp1_body(ap_ref, w1_ref, b1_ref, w2_ref, b2_ref, x2_ref):
    agg = ap_ref[0] + ap_ref[1]
    h = jax.nn.relu(
        jnp.dot(agg, w1_ref[...], preferred_element_type=jnp.float32)
        + b1_ref[...])
    x2_ref[...] = jax.nn.relu(
        jnp.dot(h, w2_ref[...], preferred_element_type=jnp.float32)
        + b2_ref[...])


def _head_body(ap_ref, wp1_ref, bp1_ref, wp2_ref, bp2_ref,
               wv1_ref, bv1_ref, wv2_ref, bv2_ref, pol_ref, val_ref):
    agg = ap_ref[0] + ap_ref[1]
    hp = jax.nn.relu(
        jnp.dot(agg, wp1_ref[...], preferred_element_type=jnp.float32)
        + bp1_ref[0])
    pol = hp * wp2_ref[0, 0] + bp2_ref[0]
    hv = jax.nn.relu(
        jnp.dot(agg, wv1_ref[...], preferred_element_type=jnp.float32)
        + bv1_ref[0])
    val = hv * wv2_ref[0, 0] + bv2_ref[0]
    ex = jnp.exp(pol - jnp.max(pol))
    pol_ref[...] = ex / jnp.sum(ex)
    m = jnp.sum(val) / N
    std = jnp.sqrt(jnp.sum((val - m) ** 2) / N + 1e-10)
    val_ref[...] = (val - m) / std


def kernel(edge_index, W1_0, b1_0, W2_0, b2_0, W1_1, b1_1, W2_1, b2_1,
           Wp1, bp1, Wp2, bp2, Wv1, bv1, Wv2, bv2):
    row = edge_index[0]
    col = edge_index[1]
    half = jnp.full((N, 1), 0.5, jnp.float32)
    ones_w = jnp.ones((K, 1), jnp.float32)
    eye8 = jnp.eye(8, dtype=jnp.float32)
    spread = jnp.repeat(eye8, F, axis=1)
    w1t = jnp.tile(W1_0[0], 8)
    b1t = jnp.tile(b1_0, 8)
    w2bd = jnp.kron(eye8, W2_0)
    b2t = jnp.tile(b2_0, 8)
    w11bd = jnp.kron(eye8, W1_1)
    b11t = jnp.tile(b1_1, 8)
    w21bd = jnp.kron(eye8, W2_1)
    b21t = jnp.tile(b2_1, 8)
    wp1bd = jnp.kron(eye8, Wp1)
    wv1bd = jnp.kron(eye8, Wv1)

    degp = _deg_kernel(half, ones_w, row)
    dp8 = jnp.reshape(degp, (NC, NP, 8))

    x1p = pl.pallas_call(
        _mlp0_body,
        out_shape=jax.ShapeDtypeStruct((NP, 128), jnp.float32),
    )(dp8, spread, w1t, b1t, w2bd, b2t)
    x1 = jnp.reshape(x1p, (N, F))

    aggp1 = _agg_kernel(x1, x1 * 0.5, row, col)
    x2p = pl.pallas_call(
        _mlp1_body,
        out_shape=jax.ShapeDtypeStruct((NP, 128), jnp.float32),
    )(jnp.reshape(aggp1, (NC, NP, 128)), w11bd, b11t, w21bd, b21t)
    x2 = jnp.reshape(x2p, (N, F))

    aggp2 = _agg_kernel(x2, x2 * 0.5, row, col)
    pol, val = pl.pallas_call(
        _head_body,
        out_shape=(jax.ShapeDtypeStruct((NP, 8), jnp.float32),
                   jax.ShapeDtypeStruct((NP, 8), jnp.float32)),
    )(jnp.reshape(aggp2, (NC, NP, 128)), wp1bd, bp1, Wp2, bp2,
      wv1bd, bv1, Wv2, bv2)

    return (jnp.reshape(pol, (N,)), jnp.reshape(val, (N,)))


# speedup vs baseline: 56.9836x; 1.0018x over previous
import functools

import jax
import jax.numpy as jnp
from jax import lax
from jax.experimental import pallas as pl
from jax.experimental.pallas import tpu as pltpu
from jax.experimental.pallas import tpu_sc as plsc

N = 100000
E = 3200000
F = 16
NC = 2
NS = 16
NW = NC * NS
EPW = E // NW
K = 4000
NCHUNK = EPW // K
KA = 800
NCHUNKA = EPW // KA
RPT = N // NS
DEG_T = 10
DEG_RPT = N // DEG_T
NP = N // 8

_mesh = plsc.VectorSubcoreMesh(core_axis_name="c", subcore_axis_name="s")
_sc_params = pltpu.CompilerParams(use_tc_tiling_on_sc=False)


@functools.partial(
    pl.kernel,
    out_type=jax.ShapeDtypeStruct((NC * N,), jnp.float32),
    mesh=_mesh,
    compiler_params=_sc_params,
    scratch_types=[
        pltpu.VMEM((K,), jnp.int32),
        pltpu.VMEM((K,), jnp.float32),
        pltpu.VMEM_SHARED((N,), jnp.float32),
        pltpu.SemaphoreType.DMA,
    ],
)
def _deg_kernel(half_hbm, ones_hbm, ei_hbm, out_hbm, row_v, ones_v, acc, sem):
    c = lax.axis_index("c")
    s = lax.axis_index("s")
    wid = s * NC + c
    ebase = wid * EPW
    @pl.when(s < DEG_T)
    def _():
        pltpu.sync_copy(half_hbm.at[pl.ds(s * DEG_RPT, DEG_RPT)],
                        acc.at[pl.ds(s * DEG_RPT, DEG_RPT)])
    pltpu.sync_copy(ones_hbm, ones_v)
    plsc.subcore_barrier()

    def chunk(i, carry):
        off = ebase + i * K
        pltpu.sync_copy(ei_hbm.at[pl.ds(off, K)], row_v)
        pltpu.sync_copy(ones_v, acc.at[row_v], add=True)
        return carry

    lax.fori_loop(0, NCHUNK, chunk, 0)
    plsc.subcore_barrier()
    @pl.when(s < DEG_T)
    def _():
        pltpu.sync_copy(acc.at[pl.ds(s * DEG_RPT, DEG_RPT)],
                        out_hbm.at[pl.ds(c * N + s * DEG_RPT, DEG_RPT)])


@functools.partial(
    pl.kernel,
    out_type=jax.ShapeDtypeStruct((NC, N, F), jnp.float32),
    mesh=_mesh,
    compiler_params=_sc_params,
    scratch_types=[
        pltpu.VMEM((KA,), jnp.int32),
        pltpu.VMEM((KA,), jnp.int32),
        pltpu.VMEM((KA, F), jnp.float32),
        pltpu.VMEM_SHARED((N, F), jnp.float32),
        pltpu.SemaphoreType.DMA,
    ],
)
def _agg_kernel(x_hbm, xh_hbm, ei_hbm, out_hbm,
                col_v, row_v, rows_v, acc, semg):
    c = lax.axis_index("c")
    s = lax.axis_index("s")
    wid = s * NC + c
    ebase = wid * EPW
    pltpu.sync_copy(xh_hbm.at[pl.ds(s * RPT, RPT)],
                    acc.at[pl.ds(s * RPT, RPT)])
    plsc.subcore_barrier()

    def chunk(i, carry):
        off = ebase + i * KA
        pltpu.sync_copy(ei_hbm.at[pl.ds(E + off, KA)], col_v)
        pltpu.async_copy(x_hbm.at[col_v], rows_v, semg).wait()
        pltpu.sync_copy(ei_hbm.at[pl.ds(off, KA)], row_v)
        pltpu.sync_copy(rows_v, acc.at[row_v], add=True)
        return carry

    lax.fori_loop(0, NCHUNKA, chunk, 0)
    plsc.subcore_barrier()
    pltpu.sync_copy(acc.at[pl.ds(s * RPT, RPT)],
                    out_hbm.at[c].at[pl.ds(s * RPT, RPT)])


def _mlp0_body(dp_ref, s_ref, w1_ref, b1_ref, w2_ref, b2_ref, x1_ref):
    d8 = dp_ref[0] + dp_ref[1]
    db = jnp.dot(d8, s_ref[...], preferred_element_type=jnp.float32,
                 precision=lax.Precision.HIGHEST)
    h = jax.nn.relu(db * w1_ref[...] + b1_ref[...])
    x1_ref[...] = jax.nn.relu(
        jnp.dot(h, w2_ref[...], preferred_element_type=jnp.float32,
                precision=lax.Precision.HIGHEST)
        + b2_ref[...])


def _mlp1_body(ap_ref, w1_ref, b1_ref, w2_ref, b2_ref, x2_ref):
    agg = ap_ref[0] + ap_ref[1]
    h = jax.nn.relu(
        jnp.dot(agg, w1_ref[...], preferred_element_type=jnp.float32,
                precision=lax.Precision.HIGHEST)
        + b1_ref[...])
    x2_ref[...] = jax.nn.relu(
        jnp.dot(h, w2_ref[...], preferred_element_type=jnp.float32,
                precision=lax.Precision.HIGHEST)
        + b2_ref[...])


def _head_body(ap_ref, wp1_ref, bp1_ref, wp2_ref, bp2_ref,
               wv1_ref, bv1_ref, wv2_ref, bv2_ref, pol_ref, val_ref):
    agg = ap_ref[0] + ap_ref[1]
    hp = jax.nn.relu(
        jnp.dot(agg, wp1_ref[...], preferred_element_type=jnp.float32,
                precision=lax.Precision.HIGHEST)
        + bp1_ref[0])
    pol = hp * wp2_ref[0, 0] + bp2_ref[0]
    hv = jax.nn.relu(
        jnp.dot(agg, wv1_ref[...], preferred_element_type=jnp.float32,
                precision=lax.Precision.HIGHEST)
        + bv1_ref[0])
    val = hv * wv2_ref[0, 0] + bv2_ref[0]
    ex = jnp.exp(pol - jnp.max(pol))
    pol_ref[...] = ex / jnp.sum(ex)
    m = jnp.sum(val) / N
    std = jnp.sqrt(jnp.sum((val - m) ** 2) / N + 1e-10)
    val_ref[...] = (val - m) / std


def kernel(edge_index, W1_0, b1_0, W2_0, b2_0, W1_1, b1_1, W2_1, b2_1,
           Wp1, bp1, Wp2, bp2, Wv1, bv1, Wv2, bv2):
    ei1 = jnp.reshape(edge_index, (2 * E,))
    half = jnp.full((N,), 0.5, jnp.float32)
    ones_w = jnp.ones((K,), jnp.float32)
    eye8 = jnp.eye(8, dtype=jnp.float32)
    spread = jnp.repeat(eye8, F, axis=1)
    w1t = jnp.tile(W1_0[0], 8)
    b1t = jnp.tile(b1_0, 8)
    w2bd = jnp.kron(eye8, W2_0)
    b2t = jnp.tile(b2_0, 8)
    w11bd = jnp.kron(eye8, W1_1)
    b11t = jnp.tile(b1_1, 8)
    w21bd = jnp.kron(eye8, W2_1)
    b21t = jnp.tile(b2_1, 8)
    wp1bd = jnp.kron(eye8, Wp1)
    wv1bd = jnp.kron(eye8, Wv1)

    degp = _deg_kernel(half, ones_w, ei1)
    dp8 = jnp.reshape(degp, (NC, NP, 8))

    x1p = pl.pallas_call(
        _mlp0_body,
        out_shape=jax.ShapeDtypeStruct((NP, 128), jnp.float32),
    )(dp8, spread, w1t, b1t, w2bd, b2t)
    x1 = jnp.reshape(x1p, (N, F))

    aggp1 = _agg_kernel(x1, x1 * 0.5, ei1)
    x2p = pl.pallas_call(
        _mlp1_body,
        out_shape=jax.ShapeDtypeStruct((NP, 128), jnp.float32),
    )(jnp.reshape(aggp1, (NC, NP, 128)), w11bd, b11t, w21bd, b21t)
    x2 = jnp.reshape(x2p, (N, F))

    aggp2 = _agg_kernel(x2, x2 * 0.5, ei1)
    pol, val = pl.pallas_call(
        _head_body,
        out_shape=(jax.ShapeDtypeStruct((NP, 8), jnp.float32),
                   jax.ShapeDtypeStruct((NP, 8), jnp.float32)),
    )(jnp.reshape(aggp2, (NC, NP, 128)), wp1bd, bp1, Wp2, bp2,
      wv1bd, bv1, Wv2, bv2)

    return (jnp.reshape(pol, (N,)), jnp.reshape(val, (N,)))
